# baseline (speedup 1.0000x reference)
import functools

import jax
import jax.numpy as jnp
from jax import lax
from jax.experimental import pallas as pl

_BETA = 0.1
_W = 1.0
_C = 0.01
_NC = 9
_N = 128
_BB = 8


def _loss_kernel(kine_in_ref, class_in_ref, kine_prT_ref, class_pr_ref,
                 mu_ref, lv_ref, out_ref):
    f32 = jnp.float32
    iota_m = lax.broadcasted_iota(jnp.int32, (_BB, _N, _N), 2).astype(f32)
    iota_n = lax.broadcasted_iota(jnp.int32, (_BB, _N, _N), 1).astype(f32)
    iota_c = lax.broadcasted_iota(jnp.int32, (1, 1, _NC), 2).astype(f32)
    iota_lane = lax.broadcasted_iota(jnp.int32, (1, 128), 1).astype(f32)
    w_cls = jnp.where(iota_c == 0.0, 2.0, jnp.where(iota_c == 8.0, 100.0, 1.0))

    x3 = kine_in_ref[...]
    yt3 = kine_prT_ref[...]
    ci3 = class_in_ref[...]
    cp3 = class_pr_ref[...]

    nk = x3.shape[2]
    iota_rep_r = lax.broadcasted_iota(jnp.int32, (nk, nk * _N), 0)
    iota_rep_c = lax.broadcasted_iota(jnp.int32, (nk, nk * _N), 1)
    rep = jnp.where(iota_rep_r == iota_rep_c // _N, 1.0, 0.0)
    xb = lax.dot_general(x3.reshape(_BB * _N, nk), rep,
                         (((1,), (0,)), ((), ())),
                         precision=lax.Precision.HIGHEST,
                         preferred_element_type=f32)
    xb3 = xb.reshape(_BB, _N, nk * _N)

    d3 = None
    for k in range(nk):
        diff = xb3[:, :, k * _N:(k + 1) * _N] - yt3[:, k:k + 1, :]
        t = diff * diff
        d3 = t if d3 is None else d3 + t

    dmin1 = jnp.min(d3, axis=2, keepdims=True)
    dmin2 = jnp.min(d3, axis=1, keepdims=True)
    idx1 = jnp.min(jnp.where(d3 == dmin1, iota_m, 128.0), axis=2,
                   keepdims=True)
    idx2 = jnp.min(jnp.where(d3 == dmin2, iota_n, 128.0), axis=1,
                   keepdims=True)
    sel3 = (jnp.where(iota_m == idx1, 1.0, 0.0)
            + jnp.where(iota_n == idx2, 1.0, 0.0))

    vmax_i = jnp.max(ci3, axis=2, keepdims=True)
    lbl_i = jnp.min(jnp.where(ci3 == vmax_i, iota_c, 9.0), axis=2,
                    keepdims=True)
    ep3 = jnp.exp(cp3)
    vmax_p = jnp.max(ep3, axis=2, keepdims=True)
    lbl_p = jnp.min(jnp.where(ep3 == vmax_p, iota_c, 9.0), axis=2,
                    keepdims=True)
    cnt_i = jnp.sum(jnp.where(lbl_i == iota_c, 1.0, 0.0), axis=1,
                    keepdims=True)
    cnt_p = jnp.sum(jnp.where(lbl_p == iota_c, 1.0, 0.0), axis=1,
                    keepdims=True)
    classnum = jnp.sum(w_cls * jnp.abs(cnt_p - cnt_i), axis=2,
                       keepdims=True)

    mu2 = mu_ref[...]
    lv2 = lv_ref[...]
    kl2 = -0.5 * jnp.sum(1.0 + lv2 - mu2 * mu2 - jnp.exp(lv2), axis=1,
                         keepdims=True)

    acc = jnp.zeros((1, 128), f32)
    for j in range(_BB):
        cham_j = jnp.sum(dmin1[j]) + jnp.sum(dmin2[j])
        a = lax.dot_general(sel3[j], cp3[j], (((1,), (0,)), ((), ())),
                            preferred_element_type=f32)
        contrib_j = jnp.sum(ci3[j] * a)
        total_j = ((1.0 - _BETA) * (cham_j - _W * contrib_j
                                    + _C * classnum[j, 0, 0])
                   + _BETA * kl2[j, 0])
        acc = acc + jnp.where(iota_lane == j, total_j, 0.0)
    out_ref[...] = acc[:, :_BB][None]


@jax.jit
def kernel(kine_input, class_input, kine_pred, class_pred, mu, log_var):
    B, N, D = kine_input.shape
    NC = class_input.shape[-1]
    LAT = mu.shape[-1]
    kine_predT = kine_pred.transpose(0, 2, 1)
    grid = (B // _BB,)
    out = pl.pallas_call(
        _loss_kernel,
        grid=grid,
        in_specs=[
            pl.BlockSpec((_BB, N, D), lambda i: (i, 0, 0)),
            pl.BlockSpec((_BB, N, NC), lambda i: (i, 0, 0)),
            pl.BlockSpec((_BB, D, N), lambda i: (i, 0, 0)),
            pl.BlockSpec((_BB, N, NC), lambda i: (i, 0, 0)),
            pl.BlockSpec((_BB, LAT), lambda i: (i, 0)),
            pl.BlockSpec((_BB, LAT), lambda i: (i, 0)),
        ],
        out_specs=pl.BlockSpec((1, 1, _BB), lambda i: (i, 0, 0)),
        out_shape=jax.ShapeDtypeStruct((B // _BB, 1, _BB), jnp.float32),
    )(kine_input, class_input, kine_predT, class_pred, mu, log_var)
    return out.reshape(B)


# speedup vs baseline: 1.3377x; 1.3377x over previous
import functools

import jax
import jax.numpy as jnp
from jax import lax
from jax.experimental import pallas as pl

_BETA = 0.1
_W = 1.0
_C = 0.01
_NC = 9
_N = 128
_BB = 16


def _loss_kernel(kine_in_ref, class_in_ref, kine_prT_ref, class_pr_ref,
                 mu_ref, lv_ref, out_ref):
    f32 = jnp.float32
    iota_m = lax.broadcasted_iota(jnp.int32, (_BB, _N, _N), 2).astype(f32)
    iota_n = lax.broadcasted_iota(jnp.int32, (_BB, _N, _N), 1).astype(f32)
    iota_c = lax.broadcasted_iota(jnp.int32, (1, 1, _NC), 2).astype(f32)
    iota_lane = lax.broadcasted_iota(jnp.int32, (1, 128), 1).astype(f32)
    w_cls = jnp.where(iota_c == 0.0, 2.0, jnp.where(iota_c == 8.0, 100.0, 1.0))

    x3 = kine_in_ref[...]
    yt3 = kine_prT_ref[...]
    ci3 = class_in_ref[...]
    cp3 = class_pr_ref[...]

    d3 = None
    for k in range(x3.shape[2]):
        diff = x3[:, :, k:k + 1] - yt3[:, k:k + 1, :]
        t = diff * diff
        d3 = t if d3 is None else d3 + t

    dmin1 = jnp.min(d3, axis=2, keepdims=True)
    dmin2 = jnp.min(d3, axis=1, keepdims=True)
    idx1 = jnp.min(jnp.where(d3 == dmin1, iota_m, 128.0), axis=2,
                   keepdims=True)
    idx2 = jnp.min(jnp.where(d3 == dmin2, iota_n, 128.0), axis=1,
                   keepdims=True)
    sel3 = (jnp.where(iota_m == idx1, 1.0, 0.0)
            + jnp.where(iota_n == idx2, 1.0, 0.0))

    vmax_i = jnp.max(ci3, axis=2, keepdims=True)
    lbl_i = jnp.min(jnp.where(ci3 == vmax_i, iota_c, 9.0), axis=2,
                    keepdims=True)
    ep3 = jnp.exp(cp3)
    vmax_p = jnp.max(ep3, axis=2, keepdims=True)
    lbl_p = jnp.min(jnp.where(ep3 == vmax_p, iota_c, 9.0), axis=2,
                    keepdims=True)
    cnt_i = jnp.sum(jnp.where(lbl_i == iota_c, 1.0, 0.0), axis=1,
                    keepdims=True)
    cnt_p = jnp.sum(jnp.where(lbl_p == iota_c, 1.0, 0.0), axis=1,
                    keepdims=True)
    classnum = jnp.sum(w_cls * jnp.abs(cnt_p - cnt_i), axis=2,
                       keepdims=True)

    mu2 = mu_ref[...]
    lv2 = lv_ref[...]
    kl2 = -0.5 * jnp.sum(1.0 + lv2 - mu2 * mu2 - jnp.exp(lv2), axis=1,
                         keepdims=True)

    acc = jnp.zeros((1, 128), f32)
    for j in range(_BB):
        cham_j = jnp.sum(dmin1[j]) + jnp.sum(dmin2[j])
        a = lax.dot_general(sel3[j], cp3[j], (((1,), (0,)), ((), ())),
                            preferred_element_type=f32)
        contrib_j = jnp.sum(ci3[j] * a)
        total_j = ((1.0 - _BETA) * (cham_j - _W * contrib_j
                                    + _C * classnum[j, 0, 0])
                   + _BETA * kl2[j, 0])
        acc = acc + jnp.where(iota_lane == j, total_j, 0.0)
    out_ref[...] = acc[:, :_BB][None]


@jax.jit
def kernel(kine_input, class_input, kine_pred, class_pred, mu, log_var):
    B, N, D = kine_input.shape
    NC = class_input.shape[-1]
    LAT = mu.shape[-1]
    kine_predT = kine_pred.transpose(0, 2, 1)
    grid = (B // _BB,)
    out = pl.pallas_call(
        _loss_kernel,
        grid=grid,
        in_specs=[
            pl.BlockSpec((_BB, N, D), lambda i: (i, 0, 0)),
            pl.BlockSpec((_BB, N, NC), lambda i: (i, 0, 0)),
            pl.BlockSpec((_BB, D, N), lambda i: (i, 0, 0)),
            pl.BlockSpec((_BB, N, NC), lambda i: (i, 0, 0)),
            pl.BlockSpec((_BB, LAT), lambda i: (i, 0)),
            pl.BlockSpec((_BB, LAT), lambda i: (i, 0)),
        ],
        out_specs=pl.BlockSpec((1, 1, _BB), lambda i: (i, 0, 0)),
        out_shape=jax.ShapeDtypeStruct((B // _BB, 1, _BB), jnp.float32),
    )(kine_input, class_input, kine_predT, class_pred, mu, log_var)
    return out.reshape(B)


# speedup vs baseline: 1.3533x; 1.0116x over previous
import functools

import jax
import jax.numpy as jnp
from jax import lax
from jax.experimental import pallas as pl

_BETA = 0.1
_W = 1.0
_C = 0.01
_NC = 9
_N = 128
_BB = 32


def _loss_kernel(kine_in_ref, class_in_ref, kine_prT_ref, class_pr_ref,
                 mu_ref, lv_ref, out_ref):
    f32 = jnp.float32
    iota_m = lax.broadcasted_iota(jnp.int32, (_BB, _N, _N), 2).astype(f32)
    iota_n = lax.broadcasted_iota(jnp.int32, (_BB, _N, _N), 1).astype(f32)
    iota_c = lax.broadcasted_iota(jnp.int32, (1, 1, _NC), 2).astype(f32)
    iota_lane = lax.broadcasted_iota(jnp.int32, (1, 128), 1).astype(f32)
    w_cls = jnp.where(iota_c == 0.0, 2.0, jnp.where(iota_c == 8.0, 100.0, 1.0))

    x3 = kine_in_ref[...]
    yt3 = kine_prT_ref[...]
    ci3 = class_in_ref[...]
    cp3 = class_pr_ref[...]

    d3 = None
    for k in range(x3.shape[2]):
        diff = x3[:, :, k:k + 1] - yt3[:, k:k + 1, :]
        t = diff * diff
        d3 = t if d3 is None else d3 + t

    dmin1 = jnp.min(d3, axis=2, keepdims=True)
    dmin2 = jnp.min(d3, axis=1, keepdims=True)
    idx1 = jnp.min(jnp.where(d3 == dmin1, iota_m, 128.0), axis=2,
                   keepdims=True)
    idx2 = jnp.min(jnp.where(d3 == dmin2, iota_n, 128.0), axis=1,
                   keepdims=True)
    sel3 = (jnp.where(iota_m == idx1, 1.0, 0.0)
            + jnp.where(iota_n == idx2, 1.0, 0.0))

    vmax_i = jnp.max(ci3, axis=2, keepdims=True)
    lbl_i = jnp.min(jnp.where(ci3 == vmax_i, iota_c, 9.0), axis=2,
                    keepdims=True)
    ep3 = jnp.exp(cp3)
    vmax_p = jnp.max(ep3, axis=2, keepdims=True)
    lbl_p = jnp.min(jnp.where(ep3 == vmax_p, iota_c, 9.0), axis=2,
                    keepdims=True)
    cnt_i = jnp.sum(jnp.where(lbl_i == iota_c, 1.0, 0.0), axis=1,
                    keepdims=True)
    cnt_p = jnp.sum(jnp.where(lbl_p == iota_c, 1.0, 0.0), axis=1,
                    keepdims=True)
    classnum = jnp.sum(w_cls * jnp.abs(cnt_p - cnt_i), axis=2,
                       keepdims=True)

    mu2 = mu_ref[...]
    lv2 = lv_ref[...]
    kl2 = -0.5 * jnp.sum(1.0 + lv2 - mu2 * mu2 - jnp.exp(lv2), axis=1,
                         keepdims=True)

    acc = jnp.zeros((1, 128), f32)
    for j in range(_BB):
        cham_j = jnp.sum(dmin1[j]) + jnp.sum(dmin2[j])
        a = lax.dot_general(sel3[j], cp3[j], (((1,), (0,)), ((), ())),
                            preferred_element_type=f32)
        contrib_j = jnp.sum(ci3[j] * a)
        total_j = ((1.0 - _BETA) * (cham_j - _W * contrib_j
                                    + _C * classnum[j, 0, 0])
                   + _BETA * kl2[j, 0])
        acc = acc + jnp.where(iota_lane == j, total_j, 0.0)
    out_ref[...] = acc[:, :_BB][None]


@jax.jit
def kernel(kine_input, class_input, kine_pred, class_pred, mu, log_var):
    B, N, D = kine_input.shape
    NC = class_input.shape[-1]
    LAT = mu.shape[-1]
    kine_predT = kine_pred.transpose(0, 2, 1)
    grid = (B // _BB,)
    out = pl.pallas_call(
        _loss_kernel,
        grid=grid,
        in_specs=[
            pl.BlockSpec((_BB, N, D), lambda i: (i, 0, 0)),
            pl.BlockSpec((_BB, N, NC), lambda i: (i, 0, 0)),
            pl.BlockSpec((_BB, D, N), lambda i: (i, 0, 0)),
            pl.BlockSpec((_BB, N, NC), lambda i: (i, 0, 0)),
            pl.BlockSpec((_BB, LAT), lambda i: (i, 0)),
            pl.BlockSpec((_BB, LAT), lambda i: (i, 0)),
        ],
        out_specs=pl.BlockSpec((1, 1, _BB), lambda i: (i, 0, 0)),
        out_shape=jax.ShapeDtypeStruct((B // _BB, 1, _BB), jnp.float32),
    )(kine_input, class_input, kine_predT, class_pred, mu, log_var)
    return out.reshape(B)


# speedup vs baseline: 2.2447x; 1.6587x over previous
import functools

import jax
import jax.numpy as jnp
from jax import lax
from jax.experimental import pallas as pl

_BETA = 0.1
_W = 1.0
_C = 0.01
_NC = 9
_N = 128
_BB = 32


def _loss_kernel(kine_in_ref, class_inT_ref, kine_prT_ref, class_prT_ref,
                 mu_ref, lv_ref, out_ref):
    f32 = jnp.float32
    iota_m = lax.broadcasted_iota(jnp.int32, (_BB, _N, _N), 2).astype(f32)
    iota_n = lax.broadcasted_iota(jnp.int32, (_BB, _N, _N), 1).astype(f32)
    iota_c = lax.broadcasted_iota(jnp.int32, (1, _NC, 1), 1).astype(f32)
    iota_lane = lax.broadcasted_iota(jnp.int32, (1, 128), 1).astype(f32)
    w_cls = jnp.where(iota_c == 0.0, 2.0, jnp.where(iota_c == 8.0, 100.0, 1.0))

    x3 = kine_in_ref[...]
    yt3 = kine_prT_ref[...]
    cit3 = class_inT_ref[...]
    cpt3 = class_prT_ref[...]

    d3 = None
    for k in range(x3.shape[2]):
        diff = x3[:, :, k:k + 1] - yt3[:, k:k + 1, :]
        t = diff * diff
        d3 = t if d3 is None else d3 + t

    dmin1 = jnp.min(d3, axis=2, keepdims=True)
    dmin2 = jnp.min(d3, axis=1, keepdims=True)
    idx1 = jnp.min(jnp.where(d3 == dmin1, iota_m, 128.0), axis=2,
                   keepdims=True)
    idx2 = jnp.min(jnp.where(d3 == dmin2, iota_n, 128.0), axis=1,
                   keepdims=True)
    sel3 = (jnp.where(iota_m == idx1, 1.0, 0.0)
            + jnp.where(iota_n == idx2, 1.0, 0.0))

    vmax_i = jnp.max(cit3, axis=1, keepdims=True)
    lbl_i = jnp.min(jnp.where(cit3 == vmax_i, iota_c, 9.0), axis=1,
                    keepdims=True)
    ep3 = jnp.exp(cpt3)
    vmax_p = jnp.max(ep3, axis=1, keepdims=True)
    lbl_p = jnp.min(jnp.where(ep3 == vmax_p, iota_c, 9.0), axis=1,
                    keepdims=True)
    cnt_i = jnp.sum(jnp.where(lbl_i == iota_c, 1.0, 0.0), axis=2,
                    keepdims=True)
    cnt_p = jnp.sum(jnp.where(lbl_p == iota_c, 1.0, 0.0), axis=2,
                    keepdims=True)
    classnum = jnp.sum(w_cls * jnp.abs(cnt_p - cnt_i), axis=1,
                       keepdims=True)

    mu2 = mu_ref[...]
    lv2 = lv_ref[...]
    kl2 = -0.5 * jnp.sum(1.0 + lv2 - mu2 * mu2 - jnp.exp(lv2), axis=1,
                         keepdims=True)

    acc = jnp.zeros((1, 128), f32)
    for j in range(_BB):
        cham_j = jnp.sum(dmin1[j]) + jnp.sum(dmin2[j])
        b = lax.dot_general(cpt3[j], sel3[j], (((1,), (1,)), ((), ())),
                            preferred_element_type=f32)
        contrib_j = jnp.sum(cit3[j] * b)
        total_j = ((1.0 - _BETA) * (cham_j - _W * contrib_j
                                    + _C * classnum[j, 0, 0])
                   + _BETA * kl2[j, 0])
        acc = acc + jnp.where(iota_lane == j, total_j, 0.0)
    out_ref[...] = acc[:, :_BB][None]


@jax.jit
def kernel(kine_input, class_input, kine_pred, class_pred, mu, log_var):
    B, N, D = kine_input.shape
    NC = class_input.shape[-1]
    LAT = mu.shape[-1]
    kine_predT = kine_pred.transpose(0, 2, 1)
    class_inputT = class_input.transpose(0, 2, 1)
    class_predT = class_pred.transpose(0, 2, 1)
    grid = (B // _BB,)
    out = pl.pallas_call(
        _loss_kernel,
        grid=grid,
        in_specs=[
            pl.BlockSpec((_BB, N, D), lambda i: (i, 0, 0)),
            pl.BlockSpec((_BB, NC, N), lambda i: (i, 0, 0)),
            pl.BlockSpec((_BB, D, N), lambda i: (i, 0, 0)),
            pl.BlockSpec((_BB, NC, N), lambda i: (i, 0, 0)),
            pl.BlockSpec((_BB, LAT), lambda i: (i, 0)),
            pl.BlockSpec((_BB, LAT), lambda i: (i, 0)),
        ],
        out_specs=pl.BlockSpec((1, 1, _BB), lambda i: (i, 0, 0)),
        out_shape=jax.ShapeDtypeStruct((B // _BB, 1, _BB), jnp.float32),
    )(kine_input, class_inputT, kine_predT, class_predT, mu, log_var)
    return out.reshape(B)


# speedup vs baseline: 2.8316x; 1.2615x over previous
import functools

import jax
import jax.numpy as jnp
from jax import lax
from jax.experimental import pallas as pl

_BETA = 0.1
_W = 1.0
_C = 0.01
_NC = 9
_N = 128
_BB = 32


def _loss_kernel(kine_in_ref, class_inT_ref, kine_prT_ref, class_prT_ref,
                 mu_ref, lv_ref, out_ref):
    f32 = jnp.float32
    iota_m = lax.broadcasted_iota(jnp.int32, (_BB, _N, _N), 2).astype(f32)
    iota_n = lax.broadcasted_iota(jnp.int32, (_BB, _N, _N), 1).astype(f32)
    iota_c = lax.broadcasted_iota(jnp.int32, (1, _NC, 1), 1).astype(f32)
    iota_lane = lax.broadcasted_iota(jnp.int32, (1, 128), 1).astype(f32)
    w_cls = jnp.where(iota_c == 0.0, 2.0, jnp.where(iota_c == 8.0, 100.0, 1.0))

    x3 = kine_in_ref[...]
    yt3 = kine_prT_ref[...]
    cit3 = class_inT_ref[...]
    cpt3 = class_prT_ref[...]

    nk = x3.shape[2]
    x2 = x3.reshape(_BB * _N, nk)
    iota_r4 = lax.broadcasted_iota(jnp.int32, (nk, _N), 0)
    d3 = None
    for k in range(nk):
        sel_k = jnp.where(iota_r4 == k, 1.0, 0.0)
        xbk = lax.dot_general(x2, sel_k, (((1,), (0,)), ((), ())),
                              preferred_element_type=f32)
        diff = xbk.reshape(_BB, _N, _N) - yt3[:, k:k + 1, :]
        t = diff * diff
        d3 = t if d3 is None else d3 + t

    dmin1 = jnp.min(d3, axis=2, keepdims=True)
    dmin2 = jnp.min(d3, axis=1, keepdims=True)
    idx1 = jnp.min(jnp.where(d3 == dmin1, iota_m, 128.0), axis=2,
                   keepdims=True)
    idx2 = jnp.min(jnp.where(d3 == dmin2, iota_n, 128.0), axis=1,
                   keepdims=True)
    sel3 = (jnp.where(iota_m == idx1, 1.0, 0.0)
            + jnp.where(iota_n == idx2, 1.0, 0.0))

    vmax_i = jnp.max(cit3, axis=1, keepdims=True)
    lbl_i = jnp.min(jnp.where(cit3 == vmax_i, iota_c, 9.0), axis=1,
                    keepdims=True)
    ep3 = jnp.exp(cpt3)
    vmax_p = jnp.max(ep3, axis=1, keepdims=True)
    lbl_p = jnp.min(jnp.where(ep3 == vmax_p, iota_c, 9.0), axis=1,
                    keepdims=True)
    cnt_i = jnp.sum(jnp.where(lbl_i == iota_c, 1.0, 0.0), axis=2,
                    keepdims=True)
    cnt_p = jnp.sum(jnp.where(lbl_p == iota_c, 1.0, 0.0), axis=2,
                    keepdims=True)
    classnum = jnp.sum(w_cls * jnp.abs(cnt_p - cnt_i), axis=1,
                       keepdims=True)

    mu2 = mu_ref[...]
    lv2 = lv_ref[...]
    kl2 = -0.5 * jnp.sum(1.0 + lv2 - mu2 * mu2 - jnp.exp(lv2), axis=1,
                         keepdims=True)

    acc = jnp.zeros((1, 128), f32)
    for j in range(_BB):
        cham_j = jnp.sum(dmin1[j]) + jnp.sum(dmin2[j])
        b = lax.dot_general(cpt3[j], sel3[j], (((1,), (1,)), ((), ())),
                            preferred_element_type=f32)
        contrib_j = jnp.sum(cit3[j] * b)
        total_j = ((1.0 - _BETA) * (cham_j - _W * contrib_j
                                    + _C * classnum[j, 0, 0])
                   + _BETA * kl2[j, 0])
        acc = acc + jnp.where(iota_lane == j, total_j, 0.0)
    out_ref[...] = acc[:, :_BB][None]


@jax.jit
def kernel(kine_input, class_input, kine_pred, class_pred, mu, log_var):
    B, N, D = kine_input.shape
    NC = class_input.shape[-1]
    LAT = mu.shape[-1]
    kine_predT = kine_pred.transpose(0, 2, 1)
    class_inputT = class_input.transpose(0, 2, 1)
    class_predT = class_pred.transpose(0, 2, 1)
    grid = (B // _BB,)
    out = pl.pallas_call(
        _loss_kernel,
        grid=grid,
        in_specs=[
            pl.BlockSpec((_BB, N, D), lambda i: (i, 0, 0)),
            pl.BlockSpec((_BB, NC, N), lambda i: (i, 0, 0)),
            pl.BlockSpec((_BB, D, N), lambda i: (i, 0, 0)),
            pl.BlockSpec((_BB, NC, N), lambda i: (i, 0, 0)),
            pl.BlockSpec((_BB, LAT), lambda i: (i, 0)),
            pl.BlockSpec((_BB, LAT), lambda i: (i, 0)),
        ],
        out_specs=pl.BlockSpec((1, 1, _BB), lambda i: (i, 0, 0)),
        out_shape=jax.ShapeDtypeStruct((B // _BB, 1, _BB), jnp.float32),
    )(kine_input, class_inputT, kine_predT, class_predT, mu, log_var)
    return out.reshape(B)
